# initial kernel scaffold (unmeasured)
import jax
import jax.numpy as jnp
from jax import lax
from jax.experimental import pallas as pl
from jax.experimental.pallas import tpu as pltpu

N_DEV = 8
B, SQ, D_MODEL = 2, 512, 768
HQ_TOT, DH = 64, 64
H_LOC = HQ_TOT // N_DEV
HD_LOC = H_LOC * DH
SKV = 512
BLK = 64


def kernel(x, Wq, K_ext, V_ext, Wo):
    x2 = x.reshape(B * SQ, D_MODEL)
    K2 = K_ext.reshape(B, SKV, HQ_TOT * DH)
    V2 = V_ext.reshape(B, SKV, HQ_TOT * DH)

    def body(x_ref, wq_ref, k_ref, v_ref, wo_ref, out_ref,
             kstage, vstage, kbuf, vbuf, qbuf, ctxbuf, carry,
             ksend_sems, vsend_sems, krecv_sem, vrecv_sem,
             ring_send_sems, ring_recv_sems):
        my = lax.axis_index("i")
        right = lax.rem(my + 1, N_DEV)

        barrier = pltpu.get_barrier_semaphore()

        @pl.when(my != 0)
        def _():
            pl.semaphore_signal(barrier, inc=1, device_id=(0,),
                                device_id_type=pl.DeviceIdType.MESH)

        @pl.when(my == 0)
        def _():
            pl.semaphore_wait(barrier, N_DEV - 1)
            kstage[...] = k_ref[...].astype(jnp.bfloat16)
            vstage[...] = v_ref[...].astype(jnp.bfloat16)
            for m in range(1, N_DEV):
                sl = pl.ds(m * HD_LOC, HD_LOC)
                pltpu.make_async_remote_copy(
                    src_ref=kstage.at[:, :, sl], dst_ref=kbuf,
                    send_sem=ksend_sems.at[m], recv_sem=krecv_sem,
                    device_id=(m,), device_id_type=pl.DeviceIdType.MESH,
                ).start()
                pltpu.make_async_remote_copy(
                    src_ref=vstage.at[:, :, sl], dst_ref=vbuf,
                    send_sem=vsend_sems.at[m], recv_sem=vrecv_sem,
                    device_id=(m,), device_id_type=pl.DeviceIdType.MESH,
                ).start()
            kbuf[...] = kstage[:, :, 0:HD_LOC]
            vbuf[...] = vstage[:, :, 0:HD_LOC]

        qbuf[...] = jnp.dot(
            x_ref[...].astype(jnp.bfloat16),
            wq_ref[...].astype(jnp.bfloat16),
            preferred_element_type=jnp.float32,
        ).astype(jnp.bfloat16)

        @pl.when(my != 0)
        def _():
            pltpu.make_async_remote_copy(
                src_ref=kstage.at[:, :, 0:HD_LOC], dst_ref=kbuf,
                send_sem=ksend_sems.at[0], recv_sem=krecv_sem,
                device_id=(0,), device_id_type=pl.DeviceIdType.MESH,
            ).wait_recv()
            pltpu.make_async_remote_copy(
                src_ref=vstage.at[:, :, 0:HD_LOC], dst_ref=vbuf,
                send_sem=vsend_sems.at[0], recv_sem=vrecv_sem,
                device_id=(0,), device_id_type=pl.DeviceIdType.MESH,
            ).wait_recv()

        rowb = lax.broadcasted_iota(jnp.int32, (SQ, SKV), 0) // BLK
        colb = lax.broadcasted_iota(jnp.int32, (SQ, SKV), 1) // BLK
        mask = colb <= rowb
        for b in range(B):
            for h in range(H_LOC):
                qh = qbuf[pl.ds(b * SQ, SQ), pl.ds(h * DH, DH)]
                kh = kbuf[b, :, pl.ds(h * DH, DH)]
                s = lax.dot_general(
                    qh, kh, (((1,), (1,)), ((), ())),
                    preferred_element_type=jnp.float32,
                ) * 0.125
                s = jnp.where(mask, s, jnp.float32(-1e9))
                mx = jnp.max(s, axis=-1, keepdims=True)
                w = jnp.exp(s - mx)
                w = w / jnp.sum(w, axis=-1, keepdims=True)
                vh = vbuf[b, :, pl.ds(h * DH, DH)]
                c = jnp.dot(w.astype(jnp.bfloat16), vh,
                            preferred_element_type=jnp.float32)
                ctxbuf[pl.ds(b * SQ, SQ), pl.ds(h * DH, DH)] = c.astype(
                    jnp.bfloat16)

        wo_b = wo_ref[...].astype(jnp.bfloat16)
        for b in range(B):
            p = jnp.dot(ctxbuf[pl.ds(b * SQ, SQ), :], wo_b,
                        preferred_element_type=jnp.float32)
            out_ref[b] = p
            carry[0, pl.ds(b * SQ, SQ), :] = p.astype(jnp.bfloat16)

        @pl.when(my == 0)
        def _():
            for m in range(1, N_DEV):
                sl = pl.ds(m * HD_LOC, HD_LOC)
                pltpu.make_async_remote_copy(
                    src_ref=kstage.at[:, :, sl], dst_ref=kbuf,
                    send_sem=ksend_sems.at[m], recv_sem=krecv_sem,
                    device_id=(m,), device_id_type=pl.DeviceIdType.MESH,
                ).wait_send()
                pltpu.make_async_remote_copy(
                    src_ref=vstage.at[:, :, sl], dst_ref=vbuf,
                    send_sem=vsend_sems.at[m], recv_sem=vrecv_sem,
                    device_id=(m,), device_id_type=pl.DeviceIdType.MESH,
                ).wait_send()

        for h in range(N_DEV - 1):
            s_slot, r_slot = h % 2, (h + 1) % 2
            rdma = pltpu.make_async_remote_copy(
                src_ref=carry.at[s_slot], dst_ref=carry.at[r_slot],
                send_sem=ring_send_sems.at[s_slot],
                recv_sem=ring_recv_sems.at[r_slot],
                device_id=(right,), device_id_type=pl.DeviceIdType.MESH,
            )
            rdma.start()
            rdma.wait()
            for b in range(B):
                out_ref[b] = out_ref[b] + carry[
                    r_slot, pl.ds(b * SQ, SQ), :].astype(jnp.float32)

    return pl.pallas_call(
        body,
        out_shape=jax.ShapeDtypeStruct((B, SQ, D_MODEL), jnp.float32),
        in_specs=[pl.BlockSpec(memory_space=pltpu.VMEM)] * 5,
        out_specs=pl.BlockSpec(memory_space=pltpu.VMEM),
        scratch_shapes=[
            pltpu.VMEM((B, SKV, HQ_TOT * DH), jnp.bfloat16),
            pltpu.VMEM((B, SKV, HQ_TOT * DH), jnp.bfloat16),
            pltpu.VMEM((B, SKV, HD_LOC), jnp.bfloat16),
            pltpu.VMEM((B, SKV, HD_LOC), jnp.bfloat16),
            pltpu.VMEM((B * SQ, HD_LOC), jnp.bfloat16),
            pltpu.VMEM((B * SQ, HD_LOC), jnp.bfloat16),
            pltpu.VMEM((2, B * SQ, D_MODEL), jnp.bfloat16),
            pltpu.SemaphoreType.DMA((N_DEV,)),
            pltpu.SemaphoreType.DMA((N_DEV,)),
            pltpu.SemaphoreType.DMA,
            pltpu.SemaphoreType.DMA,
            pltpu.SemaphoreType.DMA((2,)),
            pltpu.SemaphoreType.DMA((2,)),
        ],
        compiler_params=pltpu.CompilerParams(collective_id=0),
    )(x2, Wq, K2, V2, Wo)


# baseline (device time: 315801 ns/iter reference)
import jax
import jax.numpy as jnp
from jax import lax
from jax.experimental import pallas as pl
from jax.experimental.pallas import tpu as pltpu

N_DEV = 8
B, SQ, D_MODEL = 2, 512, 768
HQ_TOT, DH = 64, 64
H_LOC = HQ_TOT // N_DEV
HD_LOC = H_LOC * DH
SKV = 512
BLK = 64


def kernel(x, Wq, K_ext, V_ext, Wo):
    x2 = x.reshape(B * SQ, D_MODEL)
    K2 = K_ext.reshape(B, SKV, HQ_TOT * DH)
    V2 = V_ext.reshape(B, SKV, HQ_TOT * DH)

    def body(x_ref, wq_ref, k_ref, v_ref, wo_ref, out_ref,
             ksend, vsend, kbuf, vbuf, qbuf, ctxbuf, carry,
             ksend_sems, vsend_sems, krecv_sem, vrecv_sem,
             ring_send_sems, ring_recv_sems, credit_sem):
        my = lax.axis_index("i")
        right = lax.rem(my + 1, N_DEV)

        barrier = pltpu.get_barrier_semaphore()

        @pl.when(my != 0)
        def _():
            pl.semaphore_signal(barrier, inc=1, device_id=(0,),
                                device_id_type=pl.DeviceIdType.MESH)

        @pl.when(my == 0)
        def _():
            pl.semaphore_wait(barrier, N_DEV - 1)
            for m in range(1, N_DEV):
                slot = m % 2
                sl = pl.ds(m * HD_LOC, HD_LOC)
                if m >= 3:
                    pltpu.make_async_remote_copy(
                        src_ref=ksend.at[slot], dst_ref=kbuf,
                        send_sem=ksend_sems.at[m - 2], recv_sem=krecv_sem,
                        device_id=(m - 2,),
                        device_id_type=pl.DeviceIdType.MESH,
                    ).wait_send()
                    pltpu.make_async_remote_copy(
                        src_ref=vsend.at[slot], dst_ref=vbuf,
                        send_sem=vsend_sems.at[m - 2], recv_sem=vrecv_sem,
                        device_id=(m - 2,),
                        device_id_type=pl.DeviceIdType.MESH,
                    ).wait_send()
                ksend[slot] = k_ref[:, :, sl].astype(jnp.bfloat16)
                vsend[slot] = v_ref[:, :, sl].astype(jnp.bfloat16)
                pltpu.make_async_remote_copy(
                    src_ref=ksend.at[slot], dst_ref=kbuf,
                    send_sem=ksend_sems.at[m], recv_sem=krecv_sem,
                    device_id=(m,), device_id_type=pl.DeviceIdType.MESH,
                ).start()
                pltpu.make_async_remote_copy(
                    src_ref=vsend.at[slot], dst_ref=vbuf,
                    send_sem=vsend_sems.at[m], recv_sem=vrecv_sem,
                    device_id=(m,), device_id_type=pl.DeviceIdType.MESH,
                ).start()
            kbuf[...] = k_ref[:, :, 0:HD_LOC].astype(jnp.bfloat16)
            vbuf[...] = v_ref[:, :, 0:HD_LOC].astype(jnp.bfloat16)

        qbuf[...] = jnp.dot(
            x_ref[...].astype(jnp.bfloat16),
            wq_ref[...].astype(jnp.bfloat16),
            preferred_element_type=jnp.float32,
        ).astype(jnp.bfloat16)

        @pl.when(my != 0)
        def _():
            pltpu.make_async_remote_copy(
                src_ref=ksend.at[0], dst_ref=kbuf,
                send_sem=ksend_sems.at[0], recv_sem=krecv_sem,
                device_id=(0,), device_id_type=pl.DeviceIdType.MESH,
            ).wait_recv()
            pltpu.make_async_remote_copy(
                src_ref=vsend.at[0], dst_ref=vbuf,
                send_sem=vsend_sems.at[0], recv_sem=vrecv_sem,
                device_id=(0,), device_id_type=pl.DeviceIdType.MESH,
            ).wait_recv()

        rowb = lax.broadcasted_iota(jnp.int32, (SQ, SKV), 0) // BLK
        colb = lax.broadcasted_iota(jnp.int32, (SQ, SKV), 1) // BLK
        mask = colb <= rowb
        for b in range(B):
            for h in range(H_LOC):
                qh = qbuf[pl.ds(b * SQ, SQ), pl.ds(h * DH, DH)]
                kh = kbuf[b, :, pl.ds(h * DH, DH)]
                s = lax.dot_general(
                    qh, kh, (((1,), (1,)), ((), ())),
                    preferred_element_type=jnp.float32,
                ) * 0.125
                s = jnp.where(mask, s, jnp.float32(-1e9))
                mx = jnp.max(s, axis=-1, keepdims=True)
                w = jnp.exp(s - mx)
                w = w / jnp.sum(w, axis=-1, keepdims=True)
                vh = vbuf[b, :, pl.ds(h * DH, DH)]
                c = jnp.dot(w.astype(jnp.bfloat16), vh,
                            preferred_element_type=jnp.float32)
                ctxbuf[pl.ds(b * SQ, SQ), pl.ds(h * DH, DH)] = c.astype(
                    jnp.bfloat16)

        wo_b = wo_ref[...].astype(jnp.bfloat16)
        for b in range(B):
            p = jnp.dot(ctxbuf[pl.ds(b * SQ, SQ), :], wo_b,
                        preferred_element_type=jnp.float32)
            out_ref[b] = p
            carry[0, pl.ds(b * SQ, SQ), :] = p.astype(jnp.bfloat16)

        @pl.when(my == 0)
        def _():
            for m in range(N_DEV - 2, N_DEV):
                slot = m % 2
                pltpu.make_async_remote_copy(
                    src_ref=ksend.at[slot], dst_ref=kbuf,
                    send_sem=ksend_sems.at[m], recv_sem=krecv_sem,
                    device_id=(m,), device_id_type=pl.DeviceIdType.MESH,
                ).wait_send()
                pltpu.make_async_remote_copy(
                    src_ref=vsend.at[slot], dst_ref=vbuf,
                    send_sem=vsend_sems.at[m], recv_sem=vrecv_sem,
                    device_id=(m,), device_id_type=pl.DeviceIdType.MESH,
                ).wait_send()

        left = lax.rem(my + N_DEV - 1, N_DEV)
        for h in range(N_DEV - 1):
            s_slot, r_slot = h % 2, (h + 1) % 2
            if h >= 1:
                pl.semaphore_wait(credit_sem, 1)
            rdma = pltpu.make_async_remote_copy(
                src_ref=carry.at[s_slot], dst_ref=carry.at[r_slot],
                send_sem=ring_send_sems.at[s_slot],
                recv_sem=ring_recv_sems.at[r_slot],
                device_id=(right,), device_id_type=pl.DeviceIdType.MESH,
            )
            rdma.start()
            rdma.wait()
            if h < N_DEV - 2:
                pl.semaphore_signal(credit_sem, inc=1, device_id=(left,),
                                    device_id_type=pl.DeviceIdType.MESH)
            for b in range(B):
                out_ref[b] = out_ref[b] + carry[
                    r_slot, pl.ds(b * SQ, SQ), :].astype(jnp.float32)

    return pl.pallas_call(
        body,
        out_shape=jax.ShapeDtypeStruct((B, SQ, D_MODEL), jnp.float32),
        in_specs=[pl.BlockSpec(memory_space=pltpu.VMEM)] * 5,
        out_specs=pl.BlockSpec(memory_space=pltpu.VMEM),
        scratch_shapes=[
            pltpu.VMEM((2, B, SKV, HD_LOC), jnp.bfloat16),
            pltpu.VMEM((2, B, SKV, HD_LOC), jnp.bfloat16),
            pltpu.VMEM((B, SKV, HD_LOC), jnp.bfloat16),
            pltpu.VMEM((B, SKV, HD_LOC), jnp.bfloat16),
            pltpu.VMEM((B * SQ, HD_LOC), jnp.bfloat16),
            pltpu.VMEM((B * SQ, HD_LOC), jnp.bfloat16),
            pltpu.VMEM((2, B * SQ, D_MODEL), jnp.bfloat16),
            pltpu.SemaphoreType.DMA((N_DEV,)),
            pltpu.SemaphoreType.DMA((N_DEV,)),
            pltpu.SemaphoreType.DMA,
            pltpu.SemaphoreType.DMA,
            pltpu.SemaphoreType.DMA((2,)),
            pltpu.SemaphoreType.DMA((2,)),
            pltpu.SemaphoreType.REGULAR,
        ],
        compiler_params=pltpu.CompilerParams(
            collective_id=0, vmem_limit_bytes=100 * 1024 * 1024),
    )(x2, Wq, K2, V2, Wo)


# device time: 228502 ns/iter; 1.3820x vs baseline; 1.3820x over previous
import jax
import jax.numpy as jnp
from jax import lax
from jax.experimental import pallas as pl
from jax.experimental.pallas import tpu as pltpu

N_DEV = 8
B, SQ, D_MODEL = 2, 512, 768
HQ_TOT, DH = 64, 64
H_LOC = HQ_TOT // N_DEV
HD_LOC = H_LOC * DH
SKV = 512
BLK = 64
ROWS = B * SQ

SCATTER_ORDER = [6, 5, 2, 7, 3, 4, 1]


def kernel(x, Wq, K_ext, V_ext, Wo):
    x2 = x.reshape(ROWS, D_MODEL)
    K2 = K_ext.reshape(B, SKV, HQ_TOT * DH)
    V2 = V_ext.reshape(B, SKV, HQ_TOT * DH)

    def body(x_ref, wq_ref, k_ref, v_ref, wo_ref, out_ref,
             ksend, vsend, kvstage, kbuf, vbuf, qbuf, ctxbuf, acc,
             rs_send0, rs_send1, rs_send2, rs_recv0, rs_recv1, rs_recv2,
             ksend_sems, vsend_sems, krecv_sem, vrecv_sem,
             local_sems, rs_ssems, rs_rsems, ag_ssems, ag_rsems):
        my = lax.axis_index("i")

        barrier = pltpu.get_barrier_semaphore()

        @pl.when(my != 0)
        def _():
            pl.semaphore_signal(barrier, inc=1, device_id=(0,),
                                device_id_type=pl.DeviceIdType.MESH)

        @pl.when(my == 0)
        def _():
            pl.semaphore_wait(barrier, N_DEV - 1)
            for idx, m in enumerate(SCATTER_ORDER):
                slot = idx % 2
                sl = pl.ds(m * HD_LOC, HD_LOC)
                kcp = pltpu.make_async_copy(
                    k_ref.at[:, :, sl], kvstage.at[0], local_sems.at[0])
                kcp.start()
                vcp = pltpu.make_async_copy(
                    v_ref.at[:, :, sl], kvstage.at[1], local_sems.at[1])
                vcp.start()
                if idx >= 2:
                    pm = SCATTER_ORDER[idx - 2]
                    pltpu.make_async_remote_copy(
                        src_ref=ksend.at[slot], dst_ref=kbuf,
                        send_sem=ksend_sems.at[pm], recv_sem=krecv_sem,
                        device_id=(pm,),
                        device_id_type=pl.DeviceIdType.MESH,
                    ).wait_send()
                    pltpu.make_async_remote_copy(
                        src_ref=vsend.at[slot], dst_ref=vbuf,
                        send_sem=vsend_sems.at[pm], recv_sem=vrecv_sem,
                        device_id=(pm,),
                        device_id_type=pl.DeviceIdType.MESH,
                    ).wait_send()
                kcp.wait()
                ksend[slot] = kvstage[0].astype(jnp.bfloat16)
                pltpu.make_async_remote_copy(
                    src_ref=ksend.at[slot], dst_ref=kbuf,
                    send_sem=ksend_sems.at[m], recv_sem=krecv_sem,
                    device_id=(m,), device_id_type=pl.DeviceIdType.MESH,
                ).start()
                vcp.wait()
                vsend[slot] = kvstage[1].astype(jnp.bfloat16)
                pltpu.make_async_remote_copy(
                    src_ref=vsend.at[slot], dst_ref=vbuf,
                    send_sem=vsend_sems.at[m], recv_sem=vrecv_sem,
                    device_id=(m,), device_id_type=pl.DeviceIdType.MESH,
                ).start()
            kcp = pltpu.make_async_copy(
                k_ref.at[:, :, 0:HD_LOC], kvstage.at[0], local_sems.at[0])
            kcp.start()
            vcp = pltpu.make_async_copy(
                v_ref.at[:, :, 0:HD_LOC], kvstage.at[1], local_sems.at[1])
            vcp.start()
            kcp.wait()
            kbuf[...] = kvstage[0].astype(jnp.bfloat16)
            vcp.wait()
            vbuf[...] = kvstage[1].astype(jnp.bfloat16)

        qbuf[...] = jnp.dot(
            x_ref[...].astype(jnp.bfloat16),
            wq_ref[...].astype(jnp.bfloat16),
            preferred_element_type=jnp.float32,
        ).astype(jnp.bfloat16)

        @pl.when(my != 0)
        def _():
            pltpu.make_async_remote_copy(
                src_ref=ksend.at[0], dst_ref=kbuf,
                send_sem=ksend_sems.at[0], recv_sem=krecv_sem,
                device_id=(0,), device_id_type=pl.DeviceIdType.MESH,
            ).wait_recv()
            pltpu.make_async_remote_copy(
                src_ref=vsend.at[0], dst_ref=vbuf,
                send_sem=vsend_sems.at[0], recv_sem=vrecv_sem,
                device_id=(0,), device_id_type=pl.DeviceIdType.MESH,
            ).wait_recv()

        rowb = lax.broadcasted_iota(jnp.int32, (SQ, SKV), 0) // BLK
        colb = lax.broadcasted_iota(jnp.int32, (SQ, SKV), 1) // BLK
        mask = colb <= rowb
        for b in range(B):
            for h in range(H_LOC):
                qh = qbuf[pl.ds(b * SQ, SQ), pl.ds(h * DH, DH)]
                kh = kbuf[b, :, pl.ds(h * DH, DH)]
                s = lax.dot_general(
                    qh, kh, (((1,), (1,)), ((), ())),
                    preferred_element_type=jnp.float32,
                ) * 0.125
                s = jnp.where(mask, s, jnp.float32(-1e9))
                mx = jnp.max(s, axis=-1, keepdims=True)
                w = jnp.exp(s - mx)
                denom = jnp.sum(w, axis=-1, keepdims=True)
                vh = vbuf[b, :, pl.ds(h * DH, DH)]
                c = jnp.dot(w.astype(jnp.bfloat16), vh,
                            preferred_element_type=jnp.float32) / denom
                ctxbuf[pl.ds(b * SQ, SQ), pl.ds(h * DH, DH)] = c.astype(
                    jnp.bfloat16)

        wo_b = wo_ref[...].astype(jnp.bfloat16)
        for b in range(B):
            acc[pl.ds(b * SQ, SQ), :] = jnp.dot(
                ctxbuf[pl.ds(b * SQ, SQ), :], wo_b,
                preferred_element_type=jnp.float32)

        @pl.when(my == 0)
        def _():
            for idx in (len(SCATTER_ORDER) - 2, len(SCATTER_ORDER) - 1):
                m = SCATTER_ORDER[idx]
                slot = idx % 2
                pltpu.make_async_remote_copy(
                    src_ref=ksend.at[slot], dst_ref=kbuf,
                    send_sem=ksend_sems.at[m], recv_sem=krecv_sem,
                    device_id=(m,), device_id_type=pl.DeviceIdType.MESH,
                ).wait_send()
                pltpu.make_async_remote_copy(
                    src_ref=vsend.at[slot], dst_ref=vbuf,
                    send_sem=vsend_sems.at[m], recv_sem=vrecv_sem,
                    device_id=(m,), device_id_type=pl.DeviceIdType.MESH,
                ).wait_send()

        bit0 = (my >> 0) & 1
        bit1 = (my >> 1) & 1
        bit2 = (my >> 2) & 1
        bits = (bit0, bit1, bit2)

        rs_send = (rs_send0, rs_send1, rs_send2)
        rs_recv = (rs_recv0, rs_recv1, rs_recv2)
        base = 0
        for r in range(3):
            half = ROWS >> (r + 1)
            partner = my ^ (1 << r)
            keep_base = base + bits[r] * half
            send_base = base + (1 - bits[r]) * half
            rs_send[r][...] = acc[pl.ds(send_base, half), :].astype(
                jnp.bfloat16)
            rdma = pltpu.make_async_remote_copy(
                src_ref=rs_send[r], dst_ref=rs_recv[r],
                send_sem=rs_ssems.at[r], recv_sem=rs_rsems.at[r],
                device_id=(partner,), device_id_type=pl.DeviceIdType.MESH,
            )
            rdma.start()
            rdma.wait()
            acc[pl.ds(keep_base, half), :] = (
                acc[pl.ds(keep_base, half), :]
                + rs_recv[r][...].astype(jnp.float32))
            base = keep_base

        for r in (2, 1, 0):
            size = ROWS >> (r + 1)
            partner = my ^ (1 << r)
            own_base = sum(bits[j] * (ROWS >> (j + 1)) for j in range(r + 1))
            rdma = pltpu.make_async_remote_copy(
                src_ref=acc.at[pl.ds(own_base, size), :],
                dst_ref=acc.at[pl.ds(own_base, size), :],
                send_sem=ag_ssems.at[r], recv_sem=ag_rsems.at[r],
                device_id=(partner,), device_id_type=pl.DeviceIdType.MESH,
            )
            rdma.start()
            rdma.wait()

        for b in range(B):
            out_ref[b] = acc[pl.ds(b * SQ, SQ), :]

    return pl.pallas_call(
        body,
        out_shape=jax.ShapeDtypeStruct((B, SQ, D_MODEL), jnp.float32),
        in_specs=[
            pl.BlockSpec(memory_space=pltpu.VMEM),
            pl.BlockSpec(memory_space=pltpu.VMEM),
            pl.BlockSpec(memory_space=pl.ANY),
            pl.BlockSpec(memory_space=pl.ANY),
            pl.BlockSpec(memory_space=pltpu.VMEM),
        ],
        out_specs=pl.BlockSpec(memory_space=pltpu.VMEM),
        scratch_shapes=[
            pltpu.VMEM((2, B, SKV, HD_LOC), jnp.bfloat16),
            pltpu.VMEM((2, B, SKV, HD_LOC), jnp.bfloat16),
            pltpu.VMEM((2, B, SKV, HD_LOC), jnp.float32),
            pltpu.VMEM((B, SKV, HD_LOC), jnp.bfloat16),
            pltpu.VMEM((B, SKV, HD_LOC), jnp.bfloat16),
            pltpu.VMEM((ROWS, HD_LOC), jnp.bfloat16),
            pltpu.VMEM((ROWS, HD_LOC), jnp.bfloat16),
            pltpu.VMEM((ROWS, D_MODEL), jnp.float32),
            pltpu.VMEM((ROWS >> 1, D_MODEL), jnp.bfloat16),
            pltpu.VMEM((ROWS >> 2, D_MODEL), jnp.bfloat16),
            pltpu.VMEM((ROWS >> 3, D_MODEL), jnp.bfloat16),
            pltpu.VMEM((ROWS >> 1, D_MODEL), jnp.bfloat16),
            pltpu.VMEM((ROWS >> 2, D_MODEL), jnp.bfloat16),
            pltpu.VMEM((ROWS >> 3, D_MODEL), jnp.bfloat16),
            pltpu.SemaphoreType.DMA((N_DEV,)),
            pltpu.SemaphoreType.DMA((N_DEV,)),
            pltpu.SemaphoreType.DMA,
            pltpu.SemaphoreType.DMA,
            pltpu.SemaphoreType.DMA((2,)),
            pltpu.SemaphoreType.DMA((3,)),
            pltpu.SemaphoreType.DMA((3,)),
            pltpu.SemaphoreType.DMA((3,)),
            pltpu.SemaphoreType.DMA((3,)),
        ],
        compiler_params=pltpu.CompilerParams(
            collective_id=0, vmem_limit_bytes=100 * 1024 * 1024),
    )(x2, Wq, K2, V2, Wo)
